# baseline (speedup 1.0000x reference)
import functools

import jax
import jax.numpy as jnp
from jax import lax
from jax.experimental import pallas as pl
from jax.experimental.pallas import tpu as pltpu
from jax.experimental.pallas import tpu_sc as plsc

_B = 32
_C = 256
_HW = 64 * 64
_S = 64
_MINPIX = 16.0
_L = 16
_P = 128
_NCHUNK = _HW // _P

_mesh = plsc.VectorSubcoreMesh(core_axis_name="c", subcore_axis_name="s")


@functools.partial(
    pl.kernel,
    mesh=_mesh,
    compiler_params=pltpu.CompilerParams(needs_layout_passes=False),
    out_type=(
        jax.ShapeDtypeStruct((_B, _S * _C), jnp.float32),
        jax.ShapeDtypeStruct((_B, _S), jnp.float32),
    ),
    scratch_types=[
        pltpu.VMEM((_HW,), jnp.int32),
        pltpu.VMEM((_P * _C,), jnp.float32),
        pltpu.VMEM((_P * _C,), jnp.float32),
        pltpu.VMEM((_S * _C,), jnp.float32),
        pltpu.VMEM((_S,), jnp.float32),
        pltpu.VMEM((_S,), jnp.float32),
        pltpu.VMEM((_S,), jnp.float32),
        pltpu.SemaphoreType.DMA,
        pltpu.SemaphoreType.DMA,
    ],
)
def _seg_pool(seg_hbm, fmt_hbm, out_hbm, mask_hbm, ids_v, buf0_v, buf1_v,
              acc_v, cnt_v, inv_v, msk_v, sem0, sem1):
    b = lax.axis_index("s") * 2 + lax.axis_index("c")

    zeros = jnp.zeros((_L,), jnp.float32)
    ones = jnp.ones((_L,), jnp.float32)

    pltpu.async_copy(fmt_hbm.at[b, pl.ds(0, _P * _C)], buf0_v, sem0)
    pltpu.sync_copy(seg_hbm.at[b], ids_v)

    @plsc.parallel_loop(0, (_S * _C) // (4 * _L))
    def _zero(i):
        base = i * (4 * _L)
        for u in range(4):
            acc_v[pl.ds(base + u * _L, _L)] = zeros
    for s4 in range(_S // _L):
        cnt_v[pl.ds(s4 * _L, _L)] = zeros

    @plsc.parallel_loop(0, _HW // _L)
    def _count(j):
        iv = ids_v[pl.ds(j * _L, _L)]
        plsc.addupdate_scatter(cnt_v, [iv], ones)

    def _process(buf, kc):
        @plsc.parallel_loop(0, _P // _L)
        def _grp(j):
            iv = ids_v[pl.ds(kc * _P + j * _L, _L)]
            for l in range(0, _L, 2):
                rows = [iv[l] * _C, iv[l + 1] * _C]
                srcs = [(j * _L + l) * _C, (j * _L + l + 1) * _C]
                vs = [buf[pl.ds(srcs[u] + cb * _L, _L)]
                      for u in range(2) for cb in range(_C // _L)]
                for u in range(2):
                    for cb in range(_C // _L):
                        plsc.addupdate(
                            acc_v.at[pl.ds(rows[u] + cb * _L, _L)],
                            vs[u * (_C // _L) + cb])

    def _chunkpair(k2, _):
        pltpu.async_copy(
            fmt_hbm.at[b, pl.ds((2 * k2 + 1) * (_P * _C), _P * _C)],
            buf1_v, sem1)
        pltpu.make_async_copy(
            fmt_hbm.at[b, pl.ds(0, _P * _C)], buf0_v, sem0).wait()
        _process(buf0_v, 2 * k2)

        @pl.when(k2 < _NCHUNK // 2 - 1)
        def _():
            pltpu.async_copy(
                fmt_hbm.at[b, pl.ds((2 * k2 + 2) * (_P * _C), _P * _C)],
                buf0_v, sem0)

        pltpu.make_async_copy(
            fmt_hbm.at[b, pl.ds(0, _P * _C)], buf1_v, sem1).wait()
        _process(buf1_v, 2 * k2 + 1)
        return 0
    lax.fori_loop(0, _NCHUNK // 2, _chunkpair, 0)

    for s4 in range(_S // _L):
        c16 = cnt_v[pl.ds(s4 * _L, _L)]
        valid = c16 >= _MINPIX
        inv_v[pl.ds(s4 * _L, _L)] = jnp.where(
            valid, 1.0 / jnp.maximum(c16, 1.0), 0.0)
        msk_v[pl.ds(s4 * _L, _L)] = jnp.where(valid, ones, zeros)

    @plsc.parallel_loop(0, _S)
    def _scale(s):
        spl = plsc.load_gather(inv_v, [jnp.zeros((_L,), jnp.int32) + s])
        for cb in range(_C // _L):
            off = s * _C + cb * _L
            acc_v[pl.ds(off, _L)] = acc_v[pl.ds(off, _L)] * spl

    pltpu.sync_copy(acc_v, out_hbm.at[b])
    pltpu.sync_copy(msk_v, mask_hbm.at[b])


def kernel(segment_ids, sam_encoder_embeddings):
    fm = sam_encoder_embeddings.reshape(_B, _C, _HW)
    fmt = fm.swapaxes(1, 2).reshape(_B, _HW * _C)
    seg = segment_ids.reshape(_B, _HW).astype(jnp.int32)
    out_flat, mask_f = _seg_pool(seg, fmt)
    emb = out_flat.reshape(_B, _S, _C)
    return emb, mask_f > 0.5


# speedup vs baseline: 1.0055x; 1.0055x over previous
import functools

import jax
import jax.numpy as jnp
from jax import lax
from jax.experimental import pallas as pl
from jax.experimental.pallas import tpu as pltpu
from jax.experimental.pallas import tpu_sc as plsc

_B = 32
_C = 256
_HW = 64 * 64
_S = 64
_MINPIX = 16.0
_L = 16
_P = 128
_NCHUNK = _HW // _P

_mesh = plsc.VectorSubcoreMesh(core_axis_name="c", subcore_axis_name="s")


@functools.partial(
    pl.kernel,
    mesh=_mesh,
    compiler_params=pltpu.CompilerParams(needs_layout_passes=False),
    out_type=(
        jax.ShapeDtypeStruct((_B, _S * _C), jnp.float32),
        jax.ShapeDtypeStruct((_B, _S), jnp.float32),
    ),
    scratch_types=[
        pltpu.VMEM((_HW,), jnp.int32),
        pltpu.VMEM((_P * _C,), jnp.float32),
        pltpu.VMEM((_P * _C,), jnp.float32),
        pltpu.VMEM((_S * _C,), jnp.float32),
        pltpu.VMEM((_S,), jnp.float32),
        pltpu.VMEM((_S,), jnp.float32),
        pltpu.VMEM((_S,), jnp.float32),
        pltpu.SemaphoreType.DMA,
        pltpu.SemaphoreType.DMA,
    ],
)
def _seg_pool(seg_hbm, fmt_hbm, out_hbm, mask_hbm, ids_v, buf0_v, buf1_v,
              acc_v, cnt_v, inv_v, msk_v, sem0, sem1):
    b = lax.axis_index("s") * 2 + lax.axis_index("c")

    zeros = jnp.zeros((_L,), jnp.float32)
    ones = jnp.ones((_L,), jnp.float32)

    pltpu.async_copy(fmt_hbm.at[b, pl.ds(0, _P * _C)], buf0_v, sem0)
    pltpu.sync_copy(seg_hbm.at[b], ids_v)

    @plsc.parallel_loop(0, (_S * _C) // (4 * _L))
    def _zero(i):
        base = i * (4 * _L)
        for u in range(4):
            acc_v[pl.ds(base + u * _L, _L)] = zeros
    for s4 in range(_S // _L):
        cnt_v[pl.ds(s4 * _L, _L)] = zeros

    @plsc.parallel_loop(0, _HW // _L)
    def _count(j):
        iv = ids_v[pl.ds(j * _L, _L)]
        plsc.addupdate_scatter(cnt_v, [iv], ones)

    def _process(buf, kc):
        @plsc.parallel_loop(0, _P // _L)
        def _grp(j):
            iv = ids_v[pl.ds(kc * _P + j * _L, _L)]
            for l in range(_L):
                row = iv[l] * _C
                src = (j * _L + l) * _C
                vs = [buf[pl.ds(src + cb * _L, _L)]
                      for cb in range(_C // _L)]
                for cb in range(_C // _L):
                    plsc.addupdate(acc_v.at[pl.ds(row + cb * _L, _L)], vs[cb])

    def _chunkpair(k2, _):
        pltpu.async_copy(
            fmt_hbm.at[b, pl.ds((2 * k2 + 1) * (_P * _C), _P * _C)],
            buf1_v, sem1)
        pltpu.make_async_copy(
            fmt_hbm.at[b, pl.ds(0, _P * _C)], buf0_v, sem0).wait()
        _process(buf0_v, 2 * k2)

        @pl.when(k2 < _NCHUNK // 2 - 1)
        def _():
            pltpu.async_copy(
                fmt_hbm.at[b, pl.ds((2 * k2 + 2) * (_P * _C), _P * _C)],
                buf0_v, sem0)

        pltpu.make_async_copy(
            fmt_hbm.at[b, pl.ds(0, _P * _C)], buf1_v, sem1).wait()
        _process(buf1_v, 2 * k2 + 1)
        return 0
    lax.fori_loop(0, _NCHUNK // 2, _chunkpair, 0)

    for s4 in range(_S // _L):
        c16 = cnt_v[pl.ds(s4 * _L, _L)]
        valid = c16 >= _MINPIX
        inv_v[pl.ds(s4 * _L, _L)] = jnp.where(
            valid, 1.0 / jnp.maximum(c16, 1.0), 0.0)
        msk_v[pl.ds(s4 * _L, _L)] = jnp.where(valid, ones, zeros)

    @plsc.parallel_loop(0, _S)
    def _scale(s):
        spl = plsc.load_gather(inv_v, [jnp.zeros((_L,), jnp.int32) + s])
        for cb in range(_C // _L):
            off = s * _C + cb * _L
            acc_v[pl.ds(off, _L)] = acc_v[pl.ds(off, _L)] * spl

    pltpu.sync_copy(acc_v, out_hbm.at[b])
    pltpu.sync_copy(msk_v, mask_hbm.at[b])


def kernel(segment_ids, sam_encoder_embeddings):
    fm = sam_encoder_embeddings.reshape(_B, _C, _HW)
    fmt = fm.swapaxes(1, 2).reshape(_B, _HW * _C)
    seg = segment_ids.reshape(_B, _HW).astype(jnp.int32)
    out_flat, mask_f = _seg_pool(seg, fmt)
    emb = out_flat.reshape(_B, _S, _C)
    return emb, mask_f > 0.5
